# baseline (speedup 1.0000x reference)
import functools

import jax
import jax.numpy as jnp
from jax.experimental import pallas as pl


def _chamfer_batch_kernel(n_i, bi, a_ref, b_ref, min_a_ref, min_b_ref):
    bm = b_ref[0]
    m = bm.shape[0]
    bh = bm.astype(jnp.bfloat16)
    bhf = bh.astype(jnp.float32)
    b2 = jnp.sum(bhf * bhf, axis=1, keepdims=True)
    b2_hi = b2.astype(jnp.bfloat16)
    b2_lo = (b2 - b2_hi.astype(jnp.float32)).astype(jnp.bfloat16)
    bs = (-2.0 * bhf).astype(jnp.bfloat16)
    ones_b = jnp.ones((m, 2), jnp.bfloat16)
    bm_aug = jnp.concatenate([bs, ones_b, b2_hi, b2_lo], axis=1)

    def step(i, colmin):
        a = a_ref[0, i * bi:(i + 1) * bi, :]
        ah = a.astype(jnp.bfloat16)
        ahf = ah.astype(jnp.float32)
        a2 = jnp.sum(ahf * ahf, axis=1, keepdims=True)
        a2_hi = a2.astype(jnp.bfloat16)
        a2_lo = (a2 - a2_hi.astype(jnp.float32)).astype(jnp.bfloat16)
        ones_a = jnp.ones((bi, 2), jnp.bfloat16)
        a_aug = jnp.concatenate([ah, a2_hi, a2_lo, ones_a], axis=1)
        d2 = jax.lax.dot_general(
            a_aug, bm_aug, (((1,), (1,)), ((), ())),
            preferred_element_type=jnp.float32,
        )
        rowmin = jnp.min(d2, axis=1, keepdims=True)
        min_a_ref[0, i * bi:(i + 1) * bi, :] = jnp.sqrt(jnp.maximum(rowmin, 0.0))
        return jnp.minimum(colmin, jnp.min(d2, axis=0)) if colmin is not None \
            else jnp.min(d2, axis=0)

    colmin = None
    for i in range(n_i):
        colmin = step(i, colmin)
    min_b_ref[0, 0, :] = jnp.sqrt(jnp.maximum(colmin, 0.0))


def kernel(A, B):
    bt, n, d = A.shape
    m = B.shape[1]
    bi = 256
    n_i = n // bi

    min_a, min_b = pl.pallas_call(
        functools.partial(_chamfer_batch_kernel, n_i, bi),
        grid=(bt,),
        in_specs=[
            pl.BlockSpec((1, n, d), lambda b: (b, 0, 0)),
            pl.BlockSpec((1, m, d), lambda b: (b, 0, 0)),
        ],
        out_specs=[
            pl.BlockSpec((1, n, 1), lambda b: (b, 0, 0)),
            pl.BlockSpec((1, 1, m), lambda b: (b, 0, 0)),
        ],
        out_shape=[
            jax.ShapeDtypeStruct((bt, n, 1), jnp.float32),
            jax.ShapeDtypeStruct((bt, 1, m), jnp.float32),
        ],
    )(A, B)
    min_a = min_a.reshape(bt, n)
    min_b = min_b.reshape(bt, m)
    chamfer = jnp.mean(min_a, axis=1) + jnp.mean(min_b, axis=1)
    return jnp.mean(chamfer) / 12.8


# speedup vs baseline: 1.0287x; 1.0287x over previous
import functools

import jax
import jax.numpy as jnp
from jax.experimental import pallas as pl


def _chamfer_batch_kernel(n_i, bi, a_ref, b_ref, min_a_ref, min_b_ref):
    bm = b_ref[0]
    m = bm.shape[0]
    bh = bm.astype(jnp.bfloat16)
    bhf = bh.astype(jnp.float32)
    b2 = jnp.sum(bhf * bhf, axis=1, keepdims=True)
    b2_hi = b2.astype(jnp.bfloat16)
    b2_lo = (b2 - b2_hi.astype(jnp.float32)).astype(jnp.bfloat16)
    bs = (-2.0 * bhf).astype(jnp.bfloat16)
    ones_b = jnp.ones((m, 2), jnp.bfloat16)
    bm_aug = jnp.concatenate([bs, ones_b, b2_hi, b2_lo], axis=1)

    def step(i, colmin):
        a = a_ref[0, i * bi:(i + 1) * bi, :]
        ah = a.astype(jnp.bfloat16)
        ahf = ah.astype(jnp.float32)
        a2 = jnp.sum(ahf * ahf, axis=1, keepdims=True)
        a2_hi = a2.astype(jnp.bfloat16)
        a2_lo = (a2 - a2_hi.astype(jnp.float32)).astype(jnp.bfloat16)
        ones_a = jnp.ones((bi, 2), jnp.bfloat16)
        a_aug = jnp.concatenate([ah, a2_hi, a2_lo, ones_a], axis=1)
        d2 = jax.lax.dot_general(
            a_aug, bm_aug, (((1,), (1,)), ((), ())),
            preferred_element_type=jnp.float32,
        )
        rowmin = jnp.min(d2, axis=1, keepdims=True)
        min_a_ref[0, i * bi:(i + 1) * bi, :] = jnp.sqrt(jnp.maximum(rowmin, 0.0))
        return jnp.minimum(colmin, jnp.min(d2, axis=0)) if colmin is not None \
            else jnp.min(d2, axis=0)

    colmin = None
    for i in range(n_i):
        colmin = step(i, colmin)
    min_b_ref[0, 0, :] = jnp.sqrt(jnp.maximum(colmin, 0.0))


def kernel(A, B):
    bt, n, d = A.shape
    m = B.shape[1]
    bi = 1024
    n_i = n // bi

    min_a, min_b = pl.pallas_call(
        functools.partial(_chamfer_batch_kernel, n_i, bi),
        grid=(bt,),
        in_specs=[
            pl.BlockSpec((1, n, d), lambda b: (b, 0, 0)),
            pl.BlockSpec((1, m, d), lambda b: (b, 0, 0)),
        ],
        out_specs=[
            pl.BlockSpec((1, n, 1), lambda b: (b, 0, 0)),
            pl.BlockSpec((1, 1, m), lambda b: (b, 0, 0)),
        ],
        out_shape=[
            jax.ShapeDtypeStruct((bt, n, 1), jnp.float32),
            jax.ShapeDtypeStruct((bt, 1, m), jnp.float32),
        ],
    )(A, B)
    min_a = min_a.reshape(bt, n)
    min_b = min_b.reshape(bt, m)
    chamfer = jnp.mean(min_a, axis=1) + jnp.mean(min_b, axis=1)
    return jnp.mean(chamfer) / 12.8


# speedup vs baseline: 1.0371x; 1.0082x over previous
import functools

import jax
import jax.numpy as jnp
from jax.experimental import pallas as pl


def _chamfer_batch_kernel(n_i, bi, a_ref, b_ref, min_a_ref, min_b_ref):
    bm = b_ref[0]
    m = bm.shape[0]
    bh = bm.astype(jnp.bfloat16)
    bhf = bh.astype(jnp.float32)
    b2 = jnp.sum(bhf * bhf, axis=1, keepdims=True)
    b2_hi = b2.astype(jnp.bfloat16)
    b2_lo = (b2 - b2_hi.astype(jnp.float32)).astype(jnp.bfloat16)
    bs = (-2.0 * bhf).astype(jnp.bfloat16)
    ones_b = jnp.ones((m, 2), jnp.bfloat16)
    bm_aug = jnp.concatenate([bs, ones_b, b2_hi, b2_lo], axis=1)

    def step(i, colmin):
        a = a_ref[0, i * bi:(i + 1) * bi, :]
        ah = a.astype(jnp.bfloat16)
        ahf = ah.astype(jnp.float32)
        a2 = jnp.sum(ahf * ahf, axis=1, keepdims=True)
        a2_hi = a2.astype(jnp.bfloat16)
        a2_lo = (a2 - a2_hi.astype(jnp.float32)).astype(jnp.bfloat16)
        ones_a = jnp.ones((bi, 2), jnp.bfloat16)
        a_aug = jnp.concatenate([ah, a2_hi, a2_lo, ones_a], axis=1)
        d2 = jax.lax.dot_general(
            a_aug, bm_aug, (((1,), (1,)), ((), ())),
            preferred_element_type=jnp.float32,
        )
        rowmin = jnp.min(d2, axis=1, keepdims=True)
        min_a_ref[0, i * bi:(i + 1) * bi, :] = jnp.sqrt(jnp.maximum(rowmin, 0.0))
        return jnp.minimum(colmin, jnp.min(d2, axis=0)) if colmin is not None \
            else jnp.min(d2, axis=0)

    colmin = None
    for i in range(n_i):
        colmin = step(i, colmin)
    min_b_ref[0, 0, :] = jnp.sqrt(jnp.maximum(colmin, 0.0))


def kernel(A, B):
    bt, n, d = A.shape
    m = B.shape[1]
    bi = 2048
    n_i = n // bi

    min_a, min_b = pl.pallas_call(
        functools.partial(_chamfer_batch_kernel, n_i, bi),
        grid=(bt,),
        in_specs=[
            pl.BlockSpec((1, n, d), lambda b: (b, 0, 0)),
            pl.BlockSpec((1, m, d), lambda b: (b, 0, 0)),
        ],
        out_specs=[
            pl.BlockSpec((1, n, 1), lambda b: (b, 0, 0)),
            pl.BlockSpec((1, 1, m), lambda b: (b, 0, 0)),
        ],
        out_shape=[
            jax.ShapeDtypeStruct((bt, n, 1), jnp.float32),
            jax.ShapeDtypeStruct((bt, 1, m), jnp.float32),
        ],
    )(A, B)
    min_a = min_a.reshape(bt, n)
    min_b = min_b.reshape(bt, m)
    chamfer = jnp.mean(min_a, axis=1) + jnp.mean(min_b, axis=1)
    return jnp.mean(chamfer) / 12.8


# speedup vs baseline: 1.0450x; 1.0076x over previous
import functools

import jax
import jax.numpy as jnp
from jax.experimental import pallas as pl


def _chamfer_batch_kernel(n_i, bi, a_ref, b_ref, min_a_ref, min_b_ref):
    bm = b_ref[0]
    m = bm.shape[0]
    bh = bm.astype(jnp.bfloat16)
    bhf = bh.astype(jnp.float32)
    b2 = jnp.sum(bhf * bhf, axis=1, keepdims=True)
    b2_hi = b2.astype(jnp.bfloat16)
    b2_lo = (b2 - b2_hi.astype(jnp.float32)).astype(jnp.bfloat16)
    bs = (-2.0 * bhf).astype(jnp.bfloat16)
    ones_b = jnp.ones((m, 2), jnp.bfloat16)
    bm_aug = jnp.concatenate([bs, ones_b, b2_hi, b2_lo], axis=1)

    def step(i, colmin):
        a = a_ref[0, i * bi:(i + 1) * bi, :]
        ah = a.astype(jnp.bfloat16)
        ahf = ah.astype(jnp.float32)
        a2 = jnp.sum(ahf * ahf, axis=1, keepdims=True)
        a2_hi = a2.astype(jnp.bfloat16)
        a2_lo = (a2 - a2_hi.astype(jnp.float32)).astype(jnp.bfloat16)
        ones_a = jnp.ones((bi, 2), jnp.bfloat16)
        a_aug = jnp.concatenate([ah, a2_hi, a2_lo, ones_a], axis=1)
        d2 = jax.lax.dot_general(
            a_aug, bm_aug, (((1,), (1,)), ((), ())),
            preferred_element_type=jnp.float32,
        ).astype(jnp.bfloat16)
        rowmin = jnp.min(d2, axis=1, keepdims=True).astype(jnp.float32)
        min_a_ref[0, i * bi:(i + 1) * bi, :] = jnp.sqrt(jnp.maximum(rowmin, 0.0))
        return jnp.minimum(colmin, jnp.min(d2, axis=0)) if colmin is not None \
            else jnp.min(d2, axis=0)

    colmin = None
    for i in range(n_i):
        colmin = step(i, colmin)
    min_b_ref[0, 0, :] = jnp.sqrt(jnp.maximum(colmin.astype(jnp.float32), 0.0))


def kernel(A, B):
    bt, n, d = A.shape
    m = B.shape[1]
    bi = 2048
    n_i = n // bi

    min_a, min_b = pl.pallas_call(
        functools.partial(_chamfer_batch_kernel, n_i, bi),
        grid=(bt,),
        in_specs=[
            pl.BlockSpec((1, n, d), lambda b: (b, 0, 0)),
            pl.BlockSpec((1, m, d), lambda b: (b, 0, 0)),
        ],
        out_specs=[
            pl.BlockSpec((1, n, 1), lambda b: (b, 0, 0)),
            pl.BlockSpec((1, 1, m), lambda b: (b, 0, 0)),
        ],
        out_shape=[
            jax.ShapeDtypeStruct((bt, n, 1), jnp.float32),
            jax.ShapeDtypeStruct((bt, 1, m), jnp.float32),
        ],
    )(A, B)
    min_a = min_a.reshape(bt, n)
    min_b = min_b.reshape(bt, m)
    chamfer = jnp.mean(min_a, axis=1) + jnp.mean(min_b, axis=1)
    return jnp.mean(chamfer) / 12.8


# speedup vs baseline: 1.0963x; 1.0491x over previous
import functools

import jax
import jax.numpy as jnp
from jax.experimental import pallas as pl


def _chamfer_batch_kernel(n_i, bi, a_ref, b_ref, sums_ref):
    b = pl.program_id(0)
    bm = b_ref[0]
    m = bm.shape[0]
    bh = bm.astype(jnp.bfloat16)
    bhf = bh.astype(jnp.float32)
    b2 = jnp.sum(bhf * bhf, axis=1, keepdims=True)
    b2_hi = b2.astype(jnp.bfloat16)
    b2_lo = (b2 - b2_hi.astype(jnp.float32)).astype(jnp.bfloat16)
    bs = (-2.0 * bhf).astype(jnp.bfloat16)
    ones_b = jnp.ones((m, 2), jnp.bfloat16)
    bm_aug = jnp.concatenate([bs, ones_b, b2_hi, b2_lo], axis=1)

    sum_a = None
    colmin = None
    for i in range(n_i):
        a = a_ref[0, i * bi:(i + 1) * bi, :]
        ah = a.astype(jnp.bfloat16)
        ahf = ah.astype(jnp.float32)
        a2 = jnp.sum(ahf * ahf, axis=1, keepdims=True)
        a2_hi = a2.astype(jnp.bfloat16)
        a2_lo = (a2 - a2_hi.astype(jnp.float32)).astype(jnp.bfloat16)
        ones_a = jnp.ones((bi, 2), jnp.bfloat16)
        a_aug = jnp.concatenate([ah, a2_hi, a2_lo, ones_a], axis=1)
        d2 = jax.lax.dot_general(
            a_aug, bm_aug, (((1,), (1,)), ((), ())),
            preferred_element_type=jnp.float32,
        ).astype(jnp.bfloat16)
        rowmin = jnp.min(d2, axis=1, keepdims=True).astype(jnp.float32)
        dist_a = jnp.sqrt(jnp.maximum(rowmin, 0.0))
        s = jnp.sum(dist_a, keepdims=True)
        sum_a = s if sum_a is None else sum_a + s
        cm = jnp.min(d2, axis=0)
        colmin = cm if colmin is None else jnp.minimum(colmin, cm)

    dist_b = jnp.sqrt(jnp.maximum(colmin.astype(jnp.float32), 0.0))
    sum_b = jnp.sum(dist_b).reshape(1, 1)
    batch_sums = jnp.concatenate([sum_a, sum_b], axis=1)

    @pl.when(b == 0)
    def _init():
        sums_ref[0, :, :] = batch_sums

    @pl.when(b > 0)
    def _acc():
        sums_ref[0, :, :] = sums_ref[0, :, :] + batch_sums


def kernel(A, B):
    bt, n, d = A.shape
    m = B.shape[1]
    bi = 2048
    n_i = n // bi

    sums = pl.pallas_call(
        functools.partial(_chamfer_batch_kernel, n_i, bi),
        grid=(bt,),
        in_specs=[
            pl.BlockSpec((1, n, d), lambda b: (b, 0, 0)),
            pl.BlockSpec((1, m, d), lambda b: (b, 0, 0)),
        ],
        out_specs=pl.BlockSpec((1, 1, 2), lambda b: (0, 0, 0)),
        out_shape=jax.ShapeDtypeStruct((1, 1, 2), jnp.float32),
    )(A, B)
    return (sums[0, 0, 0] / n + sums[0, 0, 1] / m) / (bt * 12.8)


# speedup vs baseline: 1.1015x; 1.0047x over previous
import functools

import jax
import jax.numpy as jnp
from jax.experimental import pallas as pl


def _chamfer_batch_kernel(n_i, bi, a_ref, b_ref, sums_ref):
    b = pl.program_id(0)
    bm = b_ref[0]
    m = bm.shape[0]
    bh = bm.astype(jnp.bfloat16)
    bhf = bh.astype(jnp.float32)
    b2 = jnp.sum(bhf * bhf, axis=1, keepdims=True)
    b2_hi = b2.astype(jnp.bfloat16)
    b2_lo = (b2 - b2_hi.astype(jnp.float32)).astype(jnp.bfloat16)
    bs = (-2.0 * bhf).astype(jnp.bfloat16)
    ones_b = jnp.ones((m, 2), jnp.bfloat16)
    bm_aug = jnp.concatenate([bs, ones_b, b2_hi, b2_lo], axis=1)

    sum_a = None
    colmin = None
    for i in range(n_i):
        a = a_ref[0, i * bi:(i + 1) * bi, :]
        ah = a.astype(jnp.bfloat16)
        ahf = ah.astype(jnp.float32)
        a2 = jnp.sum(ahf * ahf, axis=1, keepdims=True)
        a2_hi = a2.astype(jnp.bfloat16)
        a2_lo = (a2 - a2_hi.astype(jnp.float32)).astype(jnp.bfloat16)
        ones_a = jnp.ones((bi, 2), jnp.bfloat16)
        a_aug = jnp.concatenate([ah, a2_hi, a2_lo, ones_a], axis=1)
        d2 = jax.lax.dot_general(
            a_aug, bm_aug, (((1,), (1,)), ((), ())),
            preferred_element_type=jnp.float32,
        ).astype(jnp.bfloat16)
        rowmin = jnp.min(d2, axis=1, keepdims=True).astype(jnp.float32)
        dist_a = jnp.sqrt(jnp.maximum(rowmin, 0.0))
        s = jnp.sum(dist_a, keepdims=True)
        sum_a = s if sum_a is None else sum_a + s
        cm = jnp.min(d2, axis=0)
        colmin = cm if colmin is None else jnp.minimum(colmin, cm)

    dist_b = jnp.sqrt(jnp.maximum(colmin.astype(jnp.float32), 0.0))
    sum_b = jnp.sum(dist_b).reshape(1, 1)
    batch_sums = jnp.concatenate([sum_a, sum_b], axis=1)

    @pl.when(b == 0)
    def _init():
        sums_ref[0, :, :] = batch_sums

    @pl.when(b > 0)
    def _acc():
        sums_ref[0, :, :] = sums_ref[0, :, :] + batch_sums


def kernel(A, B):
    bt, n, d = A.shape
    m = B.shape[1]
    bi = 4096
    n_i = n // bi

    sums = pl.pallas_call(
        functools.partial(_chamfer_batch_kernel, n_i, bi),
        grid=(bt,),
        in_specs=[
            pl.BlockSpec((1, n, d), lambda b: (b, 0, 0)),
            pl.BlockSpec((1, m, d), lambda b: (b, 0, 0)),
        ],
        out_specs=pl.BlockSpec((1, 1, 2), lambda b: (0, 0, 0)),
        out_shape=jax.ShapeDtypeStruct((1, 1, 2), jnp.float32),
    )(A, B)
    return (sums[0, 0, 0] / n + sums[0, 0, 1] / m) / (bt * 12.8)
